# initial kernel scaffold (unmeasured)
import jax
import jax.numpy as jnp
from jax import lax
from jax.experimental import pallas as pl
from jax.experimental.pallas import tpu as pltpu

N_DEV = 32


def kernel(x, w_mat):
    m_per, k = x.shape
    n_total = w_mat.shape[1]
    n_per = n_total // N_DEV
    m_total = m_per * N_DEV

    def body(x_ref, w_ref, out_ref, y_blocks, send_sems, recv_sems):
        my = lax.axis_index("i")

        y = jnp.dot(x_ref[...], w_ref[...], preferred_element_type=jnp.float32)
        for j in range(N_DEV):
            y_blocks[j] = y[:, j * n_per:(j + 1) * n_per]

        out_ref[pl.ds(my * m_per, m_per), :] = y_blocks[my, :, :]

        sends = []
        for d in range(1, N_DEV):
            j = lax.rem(my + d, N_DEV)
            rdma = pltpu.make_async_remote_copy(
                src_ref=y_blocks.at[j],
                dst_ref=out_ref.at[pl.ds(my * m_per, m_per), :],
                send_sem=send_sems.at[d - 1],
                recv_sem=recv_sems.at[d - 1],
                device_id=(j,),
                device_id_type=pl.DeviceIdType.MESH,
            )
            rdma.start()
            sends.append(rdma)

        for d in range(1, N_DEV):
            src = lax.rem(my - d + N_DEV, N_DEV)
            recv = pltpu.make_async_remote_copy(
                src_ref=y_blocks.at[0],
                dst_ref=out_ref.at[pl.ds(src * m_per, m_per), :],
                send_sem=send_sems.at[d - 1],
                recv_sem=recv_sems.at[d - 1],
                device_id=(my,),
                device_id_type=pl.DeviceIdType.MESH,
            )
            recv.wait_recv()

        for rdma in sends:
            rdma.wait_send()

    return pl.pallas_call(
        body,
        out_shape=jax.ShapeDtypeStruct((m_total, n_per), jnp.float32),
        in_specs=[
            pl.BlockSpec(memory_space=pltpu.VMEM),
            pl.BlockSpec(memory_space=pltpu.VMEM),
        ],
        out_specs=pl.BlockSpec(memory_space=pltpu.VMEM),
        scratch_shapes=[
            pltpu.VMEM((N_DEV, m_per, n_per), jnp.float32),
            pltpu.SemaphoreType.DMA((N_DEV - 1,)),
            pltpu.SemaphoreType.DMA((N_DEV - 1,)),
        ],
        compiler_params=pltpu.CompilerParams(collective_id=0),
    )(x, w_mat)


# baseline (device time: 55041 ns/iter reference)
import jax
import jax.numpy as jnp
from jax import lax
from jax.experimental import pallas as pl
from jax.experimental.pallas import tpu as pltpu

N_DEV = 32


def kernel(x, w_mat):
    m_per, k = x.shape
    n_total = w_mat.shape[1]
    n_per = n_total // N_DEV
    m_total = m_per * N_DEV

    def body(x_ref, w_ref, out_ref, y_blocks, send_sems, recv_sems):
        my = lax.axis_index("i")

        y = jnp.dot(x_ref[...], w_ref[...], preferred_element_type=jnp.float32)
        for j in range(N_DEV):
            y_blocks[j] = y[:, j * n_per:(j + 1) * n_per]

        out_ref[pl.ds(my * m_per, m_per), :] = y_blocks[my, :, :]

        sends = []
        for d in range(1, N_DEV):
            j = lax.rem(my + d, N_DEV)
            rdma = pltpu.make_async_remote_copy(
                src_ref=y_blocks.at[j],
                dst_ref=out_ref.at[pl.ds(my * m_per, m_per), :],
                send_sem=send_sems.at[d - 1],
                recv_sem=recv_sems.at[d - 1],
                device_id=(j,),
                device_id_type=pl.DeviceIdType.MESH,
            )
            rdma.start()
            sends.append(rdma)

        for d in range(1, N_DEV):
            src = lax.rem(my - d + N_DEV, N_DEV)
            recv = pltpu.make_async_remote_copy(
                src_ref=y_blocks.at[0],
                dst_ref=out_ref.at[pl.ds(src * m_per, m_per), :],
                send_sem=send_sems.at[d - 1],
                recv_sem=recv_sems.at[d - 1],
                device_id=(my,),
                device_id_type=pl.DeviceIdType.MESH,
            )
            recv.wait_recv()

        for rdma in sends:
            rdma.wait_send()

    return pl.pallas_call(
        body,
        out_shape=jax.ShapeDtypeStruct((m_total, n_per), jnp.float32),
        in_specs=[
            pl.BlockSpec(memory_space=pltpu.VMEM),
            pl.BlockSpec(memory_space=pltpu.VMEM),
        ],
        out_specs=pl.BlockSpec(memory_space=pltpu.VMEM),
        scratch_shapes=[
            pltpu.VMEM((N_DEV, m_per, n_per), jnp.float32),
            pltpu.SemaphoreType.DMA((N_DEV - 1,)),
            pltpu.SemaphoreType.DMA((N_DEV - 1,)),
        ],
        compiler_params=pltpu.CompilerParams(
            vmem_limit_bytes=100 * 1024 * 1024,
        ),
    )(x, w_mat)


# device time: 48472 ns/iter; 1.1355x vs baseline; 1.1355x over previous
import jax
import jax.numpy as jnp
from jax import lax
from jax.experimental import pallas as pl
from jax.experimental.pallas import tpu as pltpu

N_DEV = 32
GROUPS = 8
GSIZE = N_DEV // GROUPS


def kernel(x, w_mat):
    m_per, k = x.shape
    n_total = w_mat.shape[1]
    n_per = n_total // N_DEV
    gn = n_total // GROUPS
    m_total = m_per * N_DEV

    def body(x_ref, w_hbm, out_ref, xb_ref, wbuf, y_blocks,
             wdma_sems, send_sems, recv_sems):
        my = lax.axis_index("i")
        g0 = my // GSIZE

        xb_ref[...] = x_ref[...].astype(jnp.bfloat16)

        def start_wcopy(g, slot):
            jg = lax.rem(g0 + g, GROUPS)
            cp = pltpu.make_async_copy(
                w_hbm.at[:, pl.ds(jg * gn, gn)],
                wbuf.at[slot],
                wdma_sems.at[slot],
            )
            cp.start()
            return cp

        cps = [start_wcopy(0, 0)]
        for g in range(GROUPS):
            slot = g % 2
            if g + 1 < GROUPS:
                cps.append(start_wcopy(g + 1, (g + 1) % 2))
            cps[g].wait()
            wb = wbuf[slot].astype(jnp.bfloat16)
            y_grp = jnp.dot(xb_ref[...], wb,
                            preferred_element_type=jnp.float32)
            jg = lax.rem(g0 + g, GROUPS)
            for q in range(GSIZE):
                p = g * GSIZE + q
                j = jg * GSIZE + q
                d = lax.rem(j - my + N_DEV, N_DEV)
                y_blocks[p] = y_grp[:, q * n_per:(q + 1) * n_per]
                rdma = pltpu.make_async_remote_copy(
                    src_ref=y_blocks.at[p],
                    dst_ref=out_ref.at[pl.ds(my * m_per, m_per), :],
                    send_sem=send_sems.at[p],
                    recv_sem=recv_sems.at[d],
                    device_id=(j,),
                    device_id_type=pl.DeviceIdType.MESH,
                )
                rdma.start()

        for d in range(N_DEV):
            src = lax.rem(my - d + N_DEV, N_DEV)
            recv = pltpu.make_async_remote_copy(
                src_ref=y_blocks.at[0],
                dst_ref=out_ref.at[pl.ds(src * m_per, m_per), :],
                send_sem=send_sems.at[0],
                recv_sem=recv_sems.at[d],
                device_id=(my,),
                device_id_type=pl.DeviceIdType.MESH,
            )
            recv.wait_recv()

        for p in range(N_DEV):
            send = pltpu.make_async_remote_copy(
                src_ref=y_blocks.at[p],
                dst_ref=out_ref.at[pl.ds(my * m_per, m_per), :],
                send_sem=send_sems.at[p],
                recv_sem=recv_sems.at[0],
                device_id=(my,),
                device_id_type=pl.DeviceIdType.MESH,
            )
            send.wait_send()

    return pl.pallas_call(
        body,
        out_shape=jax.ShapeDtypeStruct((m_total, n_per), jnp.float32),
        in_specs=[
            pl.BlockSpec(memory_space=pltpu.VMEM),
            pl.BlockSpec(memory_space=pltpu.MemorySpace.HBM),
        ],
        out_specs=pl.BlockSpec(memory_space=pltpu.VMEM),
        scratch_shapes=[
            pltpu.VMEM((m_per, k), jnp.bfloat16),
            pltpu.VMEM((2, k, gn), jnp.float32),
            pltpu.VMEM((N_DEV, m_per, n_per), jnp.float32),
            pltpu.SemaphoreType.DMA((2,)),
            pltpu.SemaphoreType.DMA((N_DEV,)),
            pltpu.SemaphoreType.DMA((N_DEV,)),
        ],
        compiler_params=pltpu.CompilerParams(
            vmem_limit_bytes=100 * 1024 * 1024,
        ),
    )(x, w_mat)


# device time: 38109 ns/iter; 1.4443x vs baseline; 1.2719x over previous
import os

import jax
import jax.numpy as jnp
from jax import lax
from jax.experimental import pallas as pl
from jax.experimental.pallas import tpu as pltpu

try:
    _ABLATE = open(os.path.join(os.path.dirname(__file__), "ablate.txt")).read().strip()
except OSError:
    _ABLATE = ""

N_DEV = 32
GROUPS = 8
GSIZE = N_DEV // GROUPS


def kernel(x, w_mat):
    m_per, k = x.shape
    n_total = w_mat.shape[1]
    n_per = n_total // N_DEV
    gn = n_total // GROUPS
    m_total = m_per * N_DEV

    def body(x_ref, w_hbm, out_ref, xb_ref, wbuf, y_blocks,
             wdma_sems, send_sems, recv_sems):
        my = lax.axis_index("i")
        g0 = my // GSIZE

        xb_ref[...] = x_ref[...].astype(jnp.bfloat16)

        def start_wcopy(g, slot):
            jg = lax.rem(g0 + g, GROUPS)
            cp = pltpu.make_async_copy(
                w_hbm.at[:, pl.ds(jg * gn, gn)],
                wbuf.at[slot],
                wdma_sems.at[slot],
            )
            cp.start()
            return cp

        cps = [start_wcopy(0, 0)]
        for g in range(GROUPS):
            slot = g % 2
            if g + 1 < GROUPS:
                cps.append(start_wcopy(g + 1, (g + 1) % 2))
            cps[g].wait()
            wb = wbuf[slot].astype(jnp.bfloat16)
            y_grp = jnp.dot(xb_ref[...], wb,
                            preferred_element_type=jnp.float32)
            jg = lax.rem(g0 + g, GROUPS)
            for q in range(GSIZE):
                p = g * GSIZE + q
                j = jg * GSIZE + q
                d = lax.rem(j - my + N_DEV, N_DEV)
                y_blocks[p] = y_grp[:, q * n_per:(q + 1) * n_per]
                if _ABLATE == "halfmsg":
                    rdma = pltpu.make_async_remote_copy(
                        src_ref=y_blocks.at[p, pl.ds(0, m_per // 2), :],
                        dst_ref=out_ref.at[pl.ds(my * m_per, m_per // 2), :],
                        send_sem=send_sems.at[p],
                        recv_sem=recv_sems.at[d],
                        device_id=(j,),
                        device_id_type=pl.DeviceIdType.MESH,
                    )
                else:
                    rdma = pltpu.make_async_remote_copy(
                        src_ref=y_blocks.at[p],
                        dst_ref=out_ref.at[pl.ds(my * m_per, m_per), :],
                        send_sem=send_sems.at[p],
                        recv_sem=recv_sems.at[d],
                        device_id=(j,),
                        device_id_type=pl.DeviceIdType.MESH,
                    )
                rdma.start()

        recv_rows = m_per // 2 if _ABLATE == "halfmsg" else m_per
        for d in range(N_DEV):
            src = lax.rem(my - d + N_DEV, N_DEV)
            recv = pltpu.make_async_remote_copy(
                src_ref=y_blocks.at[0],
                dst_ref=out_ref.at[pl.ds(src * m_per, recv_rows), :],
                send_sem=send_sems.at[0],
                recv_sem=recv_sems.at[d],
                device_id=(my,),
                device_id_type=pl.DeviceIdType.MESH,
            )
            recv.wait_recv()

        for p in range(N_DEV):
            send = pltpu.make_async_remote_copy(
                src_ref=y_blocks.at[p, pl.ds(0, recv_rows), :],
                dst_ref=out_ref.at[pl.ds(my * m_per, recv_rows), :],
                send_sem=send_sems.at[p],
                recv_sem=recv_sems.at[0],
                device_id=(my,),
                device_id_type=pl.DeviceIdType.MESH,
            )
            send.wait_send()

    return pl.pallas_call(
        body,
        out_shape=jax.ShapeDtypeStruct((m_total, n_per), jnp.float32),
        in_specs=[
            pl.BlockSpec(memory_space=pltpu.VMEM),
            pl.BlockSpec(memory_space=pltpu.MemorySpace.HBM),
        ],
        out_specs=pl.BlockSpec(memory_space=pltpu.VMEM),
        scratch_shapes=[
            pltpu.VMEM((m_per, k), jnp.bfloat16),
            pltpu.VMEM((2, k, gn), jnp.float32),
            pltpu.VMEM((N_DEV, m_per, n_per), jnp.float32),
            pltpu.SemaphoreType.DMA((2,)),
            pltpu.SemaphoreType.DMA((N_DEV,)),
            pltpu.SemaphoreType.DMA((N_DEV,)),
        ],
        compiler_params=pltpu.CompilerParams(
            vmem_limit_bytes=100 * 1024 * 1024,
        ),
    )(x, w_mat)
